# Initial kernel scaffold; baseline (speedup 1.0000x reference)
import jax
import jax.numpy as jnp
from jax.experimental import pallas as pl


def kernel(x, edge_index, edge_attr, W1, b1, W2, b2, Wl, bl):
    raise NotImplementedError("write your pallas kernel here")


# speedup vs baseline: 6.2585x; 6.2585x over previous
import functools

import jax
import jax.numpy as jnp
from jax import lax
from jax.experimental import pallas as pl
from jax.experimental.pallas import tpu as pltpu
from jax.experimental.pallas import tpu_sc as plsc

N = 10000
E = 320000
D = 128
DE = 16
NC, NS = 2, 16
NW = NC * NS
NPAD = 10240
EPAD = 327680
G = 128
GPW = EPAD // NW // G
ROWS_PER_TILE = NPAD // NS

_mesh = functools.partial(
    plsc.VectorSubcoreMesh, core_axis_name="c", subcore_axis_name="s")

f32 = jnp.float32
i32 = jnp.int32


def _worker_id():
    return lax.axis_index("s") * NC + lax.axis_index("c")


@functools.partial(
    pl.kernel, mesh=_mesh(),
    out_type=jax.ShapeDtypeStruct((NC, NPAD), f32),
    scratch_types=[
        pltpu.VMEM((ROWS_PER_TILE,), f32),
        pltpu.VMEM((G,), f32),
        pltpu.VMEM((1, G), i32),
        pltpu.VMEM_SHARED((NPAD,), f32),
    ],
)
def _sc_degree(dst2d_hbm, out_hbm, zb, ones_v, didx, sdeg):
    c = lax.axis_index("c")
    sid = lax.axis_index("s")
    wid = _worker_id()

    def _init(i, _):
        zb[pl.ds(i * 16, 16)] = jnp.zeros((16,), f32)
        ones_v[pl.ds((i % 8) * 16, 16)] = jnp.full((16,), 1.0, f32)
        return _
    lax.fori_loop(0, ROWS_PER_TILE // 16, _init, None)

    pltpu.sync_copy(zb, sdeg.at[pl.ds(sid * ROWS_PER_TILE, ROWS_PER_TILE)])
    plsc.subcore_barrier()

    def _scat(g, _):
        row = wid * GPW + g
        pltpu.sync_copy(dst2d_hbm.at[pl.ds(row, 1)], didx)
        pltpu.sync_copy(ones_v, sdeg.at[didx.at[0]], add=True)
        return _
    lax.fori_loop(0, GPW, _scat, None)
    plsc.subcore_barrier()

    pltpu.sync_copy(sdeg.at[pl.ds(sid * ROWS_PER_TILE, ROWS_PER_TILE)], zb)
    pltpu.sync_copy(zb, out_hbm.at[c, pl.ds(sid * ROWS_PER_TILE, ROWS_PER_TILE)])


@functools.partial(
    pl.kernel, mesh=_mesh(),
    out_type=jax.ShapeDtypeStruct((NC, NPAD, D), f32),
    scratch_types=[
        pltpu.VMEM((G, D), f32),
        pltpu.VMEM((G, D), f32),
        pltpu.VMEM((G,), i32),
        pltpu.VMEM((1, G), i32),
        pltpu.VMEM_SHARED((NPAD, D), f32),
        pltpu.SemaphoreType.DMA,
    ],
)
def _sc_scatter(y_hbm, src_hbm, dst2d_hbm, out_hbm, zb, rows, sidx, didx,
                sacc, sem):
    c = lax.axis_index("c")
    sid = lax.axis_index("s")
    wid = _worker_id()

    def _zrow(i, _):
        for j in range(D // 16):
            zb[i, pl.ds(j * 16, 16)] = jnp.zeros((16,), f32)
        return _
    lax.fori_loop(0, G, _zrow, None)

    def _zacc(t, _):
        pltpu.sync_copy(zb, sacc.at[pl.ds(sid * ROWS_PER_TILE + t * G, G)])
        return _
    lax.fori_loop(0, ROWS_PER_TILE // G, _zacc, None)
    plsc.subcore_barrier()

    def _edge_group(g, _):
        ebase = wid * (GPW * G) + g * G
        pltpu.sync_copy(src_hbm.at[pl.ds(ebase, G)], sidx)
        pltpu.sync_copy(dst2d_hbm.at[pl.ds(wid * GPW + g, 1)], didx)
        pltpu.async_copy(y_hbm.at[sidx], rows, sem).wait()
        pltpu.sync_copy(rows, sacc.at[didx.at[0]], add=True)
        return _
    lax.fori_loop(0, GPW, _edge_group, None)
    plsc.subcore_barrier()

    def _wb(t, _):
        base = sid * ROWS_PER_TILE + t * G
        pltpu.sync_copy(sacc.at[pl.ds(base, G)], zb)
        pltpu.sync_copy(zb, out_hbm.at[c, pl.ds(base, G)])
        return _
    lax.fori_loop(0, ROWS_PER_TILE // G, _wb, None)


EPW = E // NW
HGRP = (EPW + G - 1) // G


@functools.partial(
    pl.kernel, mesh=_mesh(),
    out_type=jax.ShapeDtypeStruct((E * DE,), f32),
    scratch_types=[
        pltpu.VMEM((G,), i32),
        pltpu.VMEM((G,), i32),
        pltpu.VMEM((G, D), f32),
        pltpu.VMEM((G, D), f32),
        pltpu.VMEM((G * DE,), f32),
        pltpu.SemaphoreType.DMA,
    ],
)
def _sc_head(tab_hbm, src_hbm, dst_hbm, out_hbm, sidx, didx, abuf, bbuf, ob,
             sem):
    wid = _worker_id()

    def _group(g, _):
        base = wid * EPW + lax.min(g * G, EPW - G)
        pltpu.sync_copy(src_hbm.at[pl.ds(base, G)], sidx)
        pltpu.sync_copy(dst_hbm.at[pl.ds(base, G)], didx)
        pltpu.async_copy(tab_hbm.at[sidx], abuf, sem).wait()
        pltpu.async_copy(tab_hbm.at[didx], bbuf, sem).wait()

        def _add(j, _2):
            ob[pl.ds(j * DE, DE)] = abuf[j, pl.ds(0, DE)] + bbuf[j, pl.ds(DE, DE)]
            return _2
        lax.fori_loop(0, G, _add, None)
        pltpu.sync_copy(ob, out_hbm.at[pl.ds(base * DE, G * DE)])
        return _
    lax.fori_loop(0, HGRP, _group, None)


BLK = 512
GRID = NPAD // BLK


def _tc1_body(xp_ref, w1_ref, degp_ref, dinv_ref, y1_ref):
    deg = 1.0 + degp_ref[0, :] + degp_ref[1, :]
    dinv = lax.rsqrt(deg)
    dinv_ref[...] = dinv
    y1_ref[...] = jnp.dot(xp_ref[...], w1_ref[...],
                          preferred_element_type=f32) * dinv[:, None]


def _tc1(xp, w1, degp):
    return pl.pallas_call(
        _tc1_body,
        grid=(GRID,),
        in_specs=[
            pl.BlockSpec((BLK, D), lambda i: (i, 0)),
            pl.BlockSpec((D, D), lambda i: (0, 0)),
            pl.BlockSpec((NC, BLK), lambda i: (0, i)),
        ],
        out_specs=[
            pl.BlockSpec((BLK,), lambda i: (i,)),
            pl.BlockSpec((BLK, D), lambda i: (i, 0)),
        ],
        out_shape=[
            jax.ShapeDtypeStruct((NPAD,), f32),
            jax.ShapeDtypeStruct((NPAD, D), f32),
        ],
    )(xp, w1, degp)


def _tc2_body(y_ref, acc_ref, dinv_ref, b_ref, w_ref, y2_ref):
    i = pl.program_id(0)
    dinv = dinv_ref[...]
    h = (acc_ref[0] + acc_ref[1] + y_ref[...]) * dinv[:, None] + b_ref[...][None, :]
    h = jnp.maximum(h, 0.0)
    rows = lax.broadcasted_iota(i32, (BLK, 1), 0) + i * BLK
    h = jnp.where(rows < N, h, 0.0)
    y2_ref[...] = jnp.dot(h, w_ref[...], preferred_element_type=f32) * dinv[:, None]


def _tc2(y1, acc, dinv, b, w):
    return pl.pallas_call(
        _tc2_body,
        grid=(GRID,),
        in_specs=[
            pl.BlockSpec((BLK, D), lambda i: (i, 0)),
            pl.BlockSpec((NC, BLK, D), lambda i: (0, i, 0)),
            pl.BlockSpec((BLK,), lambda i: (i,)),
            pl.BlockSpec((D,), lambda i: (0,)),
            pl.BlockSpec((D, D), lambda i: (0, 0)),
        ],
        out_specs=pl.BlockSpec((BLK, D), lambda i: (i, 0)),
        out_shape=jax.ShapeDtypeStruct((NPAD, D), f32),
    )(y1, acc, dinv, b, w)


def _tc3_body(y_ref, acc_ref, dinv_ref, b_ref, wc_ref, bc_ref, tab_ref):
    i = pl.program_id(0)
    dinv = dinv_ref[...]
    h = (acc_ref[0] + acc_ref[1] + y_ref[...]) * dinv[:, None] + b_ref[...][None, :]
    h = jnp.maximum(h, 0.0)
    rows = lax.broadcasted_iota(i32, (BLK, 1), 0) + i * BLK
    h = jnp.where(rows < N, h, 0.0)
    hw = jnp.dot(h, wc_ref[...], preferred_element_type=f32) + bc_ref[...][None, :]
    tab_ref[...] = jnp.concatenate(
        [hw, jnp.zeros((BLK, D - 2 * DE), f32)], axis=1)


def _tc3(y2, acc, dinv, b, wc, bc):
    return pl.pallas_call(
        _tc3_body,
        grid=(GRID,),
        in_specs=[
            pl.BlockSpec((BLK, D), lambda i: (i, 0)),
            pl.BlockSpec((NC, BLK, D), lambda i: (0, i, 0)),
            pl.BlockSpec((BLK,), lambda i: (i,)),
            pl.BlockSpec((D,), lambda i: (0,)),
            pl.BlockSpec((D, 2 * DE), lambda i: (0, 0)),
            pl.BlockSpec((2 * DE,), lambda i: (0,)),
        ],
        out_specs=pl.BlockSpec((BLK, D), lambda i: (i, 0)),
        out_shape=jax.ShapeDtypeStruct((NPAD, D), f32),
    )(y2, acc, dinv, b, wc, bc)


def kernel(x, edge_index, edge_attr, W1, b1, W2, b2, Wl, bl):
    del edge_attr
    ei = edge_index.astype(i32)
    src, dst = ei[0], ei[1]
    pad = jnp.full((EPAD - E,), N, dtype=i32)
    srcp = jnp.concatenate([src, pad])
    dst2d = jnp.concatenate([dst, pad]).reshape(EPAD // G, G)
    xp = jnp.pad(x, ((0, NPAD - N), (0, 0)))

    degp = _sc_degree(dst2d)
    dinv, y1 = _tc1(xp, W1, degp)
    acc1 = _sc_scatter(y1, srcp, dst2d)
    y2 = _tc2(y1, acc1, dinv, b1, W2)
    acc2 = _sc_scatter(y2, srcp, dst2d)
    wc = jnp.concatenate([Wl[:D], Wl[D:]], axis=1)
    bc = jnp.concatenate([bl, jnp.zeros((DE,), f32)])
    tab = _tc3(y2, acc2, dinv, b2, wc, bc)
    out_flat = _sc_head(tab, src, dst)
    return out_flat.reshape(E, DE)


# speedup vs baseline: 8.4452x; 1.3494x over previous
import functools

import jax
import jax.numpy as jnp
from jax import lax
from jax.experimental import pallas as pl
from jax.experimental.pallas import tpu as pltpu
from jax.experimental.pallas import tpu_sc as plsc

N = 10000
E = 320000
D = 128
DE = 16
NC, NS = 2, 16
NW = NC * NS
NPAD = 10240
EPAD = 327680
G = 128
GPW = EPAD // NW // G
ROWS_PER_TILE = NPAD // NS

_mesh = functools.partial(
    plsc.VectorSubcoreMesh, core_axis_name="c", subcore_axis_name="s")

f32 = jnp.float32
i32 = jnp.int32


def _worker_id():
    return lax.axis_index("s") * NC + lax.axis_index("c")


@functools.partial(
    pl.kernel, mesh=_mesh(),
    out_type=jax.ShapeDtypeStruct((NC, NPAD), f32),
    scratch_types=[
        pltpu.VMEM((ROWS_PER_TILE,), f32),
        pltpu.VMEM((G,), f32),
        pltpu.VMEM((1, G), i32),
        pltpu.VMEM_SHARED((NPAD,), f32),
    ],
)
def _sc_degree(dst2d_hbm, out_hbm, zb, ones_v, didx, sdeg):
    c = lax.axis_index("c")
    sid = lax.axis_index("s")
    wid = _worker_id()

    def _init(i, _):
        zb[pl.ds(i * 16, 16)] = jnp.zeros((16,), f32)
        ones_v[pl.ds((i % 8) * 16, 16)] = jnp.full((16,), 1.0, f32)
        return _
    lax.fori_loop(0, ROWS_PER_TILE // 16, _init, None)

    pltpu.sync_copy(zb, sdeg.at[pl.ds(sid * ROWS_PER_TILE, ROWS_PER_TILE)])
    plsc.subcore_barrier()

    def _scat(g, _):
        row = wid * GPW + g
        pltpu.sync_copy(dst2d_hbm.at[pl.ds(row, 1)], didx)
        pltpu.sync_copy(ones_v, sdeg.at[didx.at[0]], add=True)
        return _
    lax.fori_loop(0, GPW, _scat, None)
    plsc.subcore_barrier()

    pltpu.sync_copy(sdeg.at[pl.ds(sid * ROWS_PER_TILE, ROWS_PER_TILE)], zb)
    pltpu.sync_copy(zb, out_hbm.at[c, pl.ds(sid * ROWS_PER_TILE, ROWS_PER_TILE)])


@functools.partial(
    pl.kernel, mesh=_mesh(),
    out_type=jax.ShapeDtypeStruct((NC, NPAD, D), f32),
    scratch_types=[
        pltpu.VMEM((G, D), f32),
        pltpu.VMEM((G, D), f32),
        pltpu.VMEM((GPW * G,), i32),
        pltpu.VMEM((1, G), i32),
        pltpu.VMEM((1, G), i32),
        pltpu.VMEM_SHARED((NPAD, D), f32),
        pltpu.SemaphoreType.DMA,
        pltpu.SemaphoreType.DMA,
        pltpu.SemaphoreType.DMA,
        pltpu.SemaphoreType.DMA,
    ],
)
def _sc_scatter(y_hbm, src_hbm, dst2d_hbm, out_hbm, rows0, rows1,
                sidx, didx0, didx1, sacc, sem0, sem1, dsem0, dsem1):
    c = lax.axis_index("c")
    sid = lax.axis_index("s")
    wid = _worker_id()

    def _zrow(i, _):
        for j in range(D // 16):
            rows0[i, pl.ds(j * 16, 16)] = jnp.zeros((16,), f32)
        return _
    lax.fori_loop(0, G, _zrow, None)

    def _zacc(t, _):
        pltpu.sync_copy(rows0, sacc.at[pl.ds(sid * ROWS_PER_TILE + t * G, G)])
        return _
    lax.fori_loop(0, ROWS_PER_TILE // G, _zacc, None)

    pltpu.sync_copy(src_hbm.at[pl.ds(wid * (GPW * G), GPW * G)], sidx)
    plsc.subcore_barrier()

    rows = (rows0, rows1)
    sems = (sem0, sem1)
    didx = (didx0, didx1)
    dsems = (dsem0, dsem1)

    def _issue(g, b):
        pltpu.async_copy(y_hbm.at[sidx.at[pl.ds(g * G, G)]], rows[b], sems[b])
        pltpu.async_copy(dst2d_hbm.at[pl.ds(wid * GPW + g, 1)], didx[b],
                         dsems[b])

    _issue(0, 0)

    def _pair(p, _):
        for b in range(2):
            g = p * 2 + b

            @pl.when(g + 1 < GPW)
            def _():
                _issue(g + 1, 1 - b)
            pltpu.make_async_copy(y_hbm.at[sidx.at[pl.ds(g * G, G)]],
                                  rows[b], sems[b]).wait()
            pltpu.make_async_copy(dst2d_hbm.at[pl.ds(wid * GPW + g, 1)],
                                  didx[b], dsems[b]).wait()
            pltpu.sync_copy(rows[b], sacc.at[didx[b].at[0]], add=True)
        return _
    lax.fori_loop(0, GPW // 2, _pair, None)
    plsc.subcore_barrier()

    def _wb(t, _):
        base = sid * ROWS_PER_TILE + t * G
        pltpu.sync_copy(sacc.at[pl.ds(base, G)], rows0)
        pltpu.sync_copy(rows0, out_hbm.at[c, pl.ds(base, G)])
        return _
    lax.fori_loop(0, ROWS_PER_TILE // G, _wb, None)


EPW = E // NW
HGRP = (EPW + G - 1) // G


@functools.partial(
    pl.kernel, mesh=_mesh(),
    out_type=jax.ShapeDtypeStruct((E * DE,), f32),
    scratch_types=[
        pltpu.VMEM((EPW,), i32),
        pltpu.VMEM((EPW,), i32),
        pltpu.VMEM((G, D), f32),
        pltpu.VMEM((G, D), f32),
        pltpu.VMEM((G, D), f32),
        pltpu.VMEM((G, D), f32),
        pltpu.VMEM((G * DE,), f32),
        pltpu.SemaphoreType.DMA,
        pltpu.SemaphoreType.DMA,
        pltpu.SemaphoreType.DMA,
        pltpu.SemaphoreType.DMA,
    ],
)
def _sc_head(tab_hbm, src_hbm, dst_hbm, out_hbm, sidx, didx,
             abuf0, abuf1, bbuf0, bbuf1, ob, sa0, sa1, sb0, sb1):
    wid = _worker_id()

    pltpu.sync_copy(src_hbm.at[pl.ds(wid * EPW, EPW)], sidx)
    pltpu.sync_copy(dst_hbm.at[pl.ds(wid * EPW, EPW)], didx)

    abufs, bbufs = (abuf0, abuf1), (bbuf0, bbuf1)
    sas, sbs = (sa0, sa1), (sb0, sb1)

    def _gather(g, b):
        off = lax.min(g * G, EPW - G)
        pltpu.async_copy(tab_hbm.at[sidx.at[pl.ds(off, G)]], abufs[b], sas[b])
        pltpu.async_copy(tab_hbm.at[didx.at[pl.ds(off, G)]], bbufs[b], sbs[b])

    _gather(0, 0)

    def _odd(g, b):
        off = lax.min(g * G, EPW - G)

        @pl.when(g + 1 < HGRP)
        def _():
            _gather(g + 1, 1 - b)
        pltpu.make_async_copy(tab_hbm.at[sidx.at[pl.ds(off, G)]],
                              abufs[b], sas[b]).wait()
        pltpu.make_async_copy(tab_hbm.at[didx.at[pl.ds(off, G)]],
                              bbufs[b], sbs[b]).wait()

        ab, bb = abufs[b], bbufs[b]

        def _add(j, _2):
            ob[pl.ds(j * DE, DE)] = ab[j, pl.ds(0, DE)] + bb[j, pl.ds(DE, DE)]
            return _2
        lax.fori_loop(0, G, _add, None)
        base = wid * EPW + off
        pltpu.sync_copy(ob, out_hbm.at[pl.ds(base * DE, G * DE)])

    def _pair(p, _):
        _odd(p * 2, 0)
        _odd(p * 2 + 1, 1)
        return _
    lax.fori_loop(0, HGRP // 2, _pair, None)
    _odd(HGRP - 1, (HGRP - 1) % 2)


BLK = 512
GRID = NPAD // BLK


def _tc1_body(xp_ref, w1_ref, degp_ref, dinv_ref, y1_ref):
    deg = 1.0 + degp_ref[0, :] + degp_ref[1, :]
    dinv = lax.rsqrt(deg)
    dinv_ref[...] = dinv
    y1_ref[...] = jnp.dot(xp_ref[...], w1_ref[...],
                          preferred_element_type=f32) * dinv[:, None]


def _tc1(xp, w1, degp):
    return pl.pallas_call(
        _tc1_body,
        grid=(GRID,),
        in_specs=[
            pl.BlockSpec((BLK, D), lambda i: (i, 0)),
            pl.BlockSpec((D, D), lambda i: (0, 0)),
            pl.BlockSpec((NC, BLK), lambda i: (0, i)),
        ],
        out_specs=[
            pl.BlockSpec((BLK,), lambda i: (i,)),
            pl.BlockSpec((BLK, D), lambda i: (i, 0)),
        ],
        out_shape=[
            jax.ShapeDtypeStruct((NPAD,), f32),
            jax.ShapeDtypeStruct((NPAD, D), f32),
        ],
    )(xp, w1, degp)


def _tc2_body(y_ref, acc_ref, dinv_ref, b_ref, w_ref, y2_ref):
    i = pl.program_id(0)
    dinv = dinv_ref[...]
    h = (acc_ref[0] + acc_ref[1] + y_ref[...]) * dinv[:, None] + b_ref[...][None, :]
    h = jnp.maximum(h, 0.0)
    rows = lax.broadcasted_iota(i32, (BLK, 1), 0) + i * BLK
    h = jnp.where(rows < N, h, 0.0)
    y2_ref[...] = jnp.dot(h, w_ref[...], preferred_element_type=f32) * dinv[:, None]


def _tc2(y1, acc, dinv, b, w):
    return pl.pallas_call(
        _tc2_body,
        grid=(GRID,),
        in_specs=[
            pl.BlockSpec((BLK, D), lambda i: (i, 0)),
            pl.BlockSpec((NC, BLK, D), lambda i: (0, i, 0)),
            pl.BlockSpec((BLK,), lambda i: (i,)),
            pl.BlockSpec((D,), lambda i: (0,)),
            pl.BlockSpec((D, D), lambda i: (0, 0)),
        ],
        out_specs=pl.BlockSpec((BLK, D), lambda i: (i, 0)),
        out_shape=jax.ShapeDtypeStruct((NPAD, D), f32),
    )(y1, acc, dinv, b, w)


def _tc3_body(y_ref, acc_ref, dinv_ref, b_ref, wc_ref, bc_ref, tab_ref):
    i = pl.program_id(0)
    dinv = dinv_ref[...]
    h = (acc_ref[0] + acc_ref[1] + y_ref[...]) * dinv[:, None] + b_ref[...][None, :]
    h = jnp.maximum(h, 0.0)
    rows = lax.broadcasted_iota(i32, (BLK, 1), 0) + i * BLK
    h = jnp.where(rows < N, h, 0.0)
    hw = jnp.dot(h, wc_ref[...], preferred_element_type=f32) + bc_ref[...][None, :]
    tab_ref[...] = jnp.concatenate(
        [hw, jnp.zeros((BLK, D - 2 * DE), f32)], axis=1)


def _tc3(y2, acc, dinv, b, wc, bc):
    return pl.pallas_call(
        _tc3_body,
        grid=(GRID,),
        in_specs=[
            pl.BlockSpec((BLK, D), lambda i: (i, 0)),
            pl.BlockSpec((NC, BLK, D), lambda i: (0, i, 0)),
            pl.BlockSpec((BLK,), lambda i: (i,)),
            pl.BlockSpec((D,), lambda i: (0,)),
            pl.BlockSpec((D, 2 * DE), lambda i: (0, 0)),
            pl.BlockSpec((2 * DE,), lambda i: (0,)),
        ],
        out_specs=pl.BlockSpec((BLK, D), lambda i: (i, 0)),
        out_shape=jax.ShapeDtypeStruct((NPAD, D), f32),
    )(y2, acc, dinv, b, wc, bc)


def kernel(x, edge_index, edge_attr, W1, b1, W2, b2, Wl, bl):
    del edge_attr
    ei = edge_index.astype(i32)
    src, dst = ei[0], ei[1]
    pad = jnp.full((EPAD - E,), N, dtype=i32)
    srcp = jnp.concatenate([src, pad])
    dst2d = jnp.concatenate([dst, pad]).reshape(EPAD // G, G)
    xp = jnp.pad(x, ((0, NPAD - N), (0, 0)))

    degp = _sc_degree(dst2d)
    dinv, y1 = _tc1(xp, W1, degp)
    acc1 = _sc_scatter(y1, srcp, dst2d)
    y2 = _tc2(y1, acc1, dinv, b1, W2)
    acc2 = _sc_scatter(y2, srcp, dst2d)
    wc = jnp.concatenate([Wl[:D], Wl[D:]], axis=1)
    bc = jnp.concatenate([bl, jnp.zeros((DE,), f32)])
    tab = _tc3(y2, acc2, dinv, b2, wc, bc)
    out_flat = _sc_head(tab, src, dst)
    return out_flat.reshape(E, DE)


# speedup vs baseline: 8.7078x; 1.0311x over previous
import functools

import jax
import jax.numpy as jnp
from jax import lax
from jax.experimental import pallas as pl
from jax.experimental.pallas import tpu as pltpu
from jax.experimental.pallas import tpu_sc as plsc

N = 10000
E = 320000
D = 128
DE = 16
NC, NS = 2, 16
NW = NC * NS
NPAD = 10240
EPAD = 327680
G = 128
GPW = EPAD // NW // G
ROWS_PER_TILE = NPAD // NS

_mesh = functools.partial(
    plsc.VectorSubcoreMesh, core_axis_name="c", subcore_axis_name="s")

f32 = jnp.float32
i32 = jnp.int32


def _worker_id():
    return lax.axis_index("s") * NC + lax.axis_index("c")


@functools.partial(
    pl.kernel, mesh=_mesh(),
    out_type=jax.ShapeDtypeStruct((NC, NPAD), f32),
    scratch_types=[
        pltpu.VMEM((ROWS_PER_TILE,), f32),
        pltpu.VMEM((G,), f32),
        pltpu.VMEM((GPW, G), i32),
        pltpu.VMEM_SHARED((NPAD,), f32),
    ],
)
def _sc_degree(dst2d_hbm, out_hbm, zb, ones_v, didx, sdeg):
    c = lax.axis_index("c")
    sid = lax.axis_index("s")
    wid = _worker_id()

    def _init(i, _):
        zb[pl.ds(i * 16, 16)] = jnp.zeros((16,), f32)
        ones_v[pl.ds((i % 8) * 16, 16)] = jnp.full((16,), 1.0, f32)
        return _
    lax.fori_loop(0, ROWS_PER_TILE // 16, _init, None)

    pltpu.sync_copy(zb, sdeg.at[pl.ds(sid * ROWS_PER_TILE, ROWS_PER_TILE)])
    pltpu.sync_copy(dst2d_hbm.at[pl.ds(wid * GPW, GPW)], didx)
    plsc.subcore_barrier()

    def _scat(g, _):
        pltpu.sync_copy(ones_v, sdeg.at[didx.at[g]], add=True)
        return _
    lax.fori_loop(0, GPW, _scat, None)
    plsc.subcore_barrier()

    pltpu.sync_copy(sdeg.at[pl.ds(sid * ROWS_PER_TILE, ROWS_PER_TILE)], zb)
    pltpu.sync_copy(zb, out_hbm.at[c, pl.ds(sid * ROWS_PER_TILE, ROWS_PER_TILE)])


@functools.partial(
    pl.kernel, mesh=_mesh(),
    out_type=jax.ShapeDtypeStruct((NC, NPAD, D), f32),
    scratch_types=[
        pltpu.VMEM((G, D), f32),
        pltpu.VMEM((G, D), f32),
        pltpu.VMEM((GPW * G,), i32),
        pltpu.VMEM((1, G), i32),
        pltpu.VMEM((1, G), i32),
        pltpu.VMEM_SHARED((NPAD, D), f32),
        pltpu.SemaphoreType.DMA,
        pltpu.SemaphoreType.DMA,
        pltpu.SemaphoreType.DMA,
        pltpu.SemaphoreType.DMA,
    ],
)
def _sc_scatter(y_hbm, src_hbm, dst2d_hbm, out_hbm, rows0, rows1,
                sidx, didx0, didx1, sacc, sem0, sem1, dsem0, dsem1):
    c = lax.axis_index("c")
    sid = lax.axis_index("s")
    wid = _worker_id()

    def _zrow(i, _):
        for j in range(D // 16):
            rows0[i, pl.ds(j * 16, 16)] = jnp.zeros((16,), f32)
        return _
    lax.fori_loop(0, G, _zrow, None)

    def _zacc(t, _):
        pltpu.sync_copy(rows0, sacc.at[pl.ds(sid * ROWS_PER_TILE + t * G, G)])
        return _
    lax.fori_loop(0, ROWS_PER_TILE // G, _zacc, None)

    pltpu.sync_copy(src_hbm.at[pl.ds(wid * (GPW * G), GPW * G)], sidx)
    plsc.subcore_barrier()

    rows = (rows0, rows1)
    sems = (sem0, sem1)
    didx = (didx0, didx1)
    dsems = (dsem0, dsem1)

    def _issue(g, b):
        pltpu.async_copy(y_hbm.at[sidx.at[pl.ds(g * G, G)]], rows[b], sems[b])
        pltpu.async_copy(dst2d_hbm.at[pl.ds(wid * GPW + g, 1)], didx[b],
                         dsems[b])

    _issue(0, 0)

    def _pair(p, _):
        for b in range(2):
            g = p * 2 + b

            @pl.when(g + 1 < GPW)
            def _():
                _issue(g + 1, 1 - b)
            pltpu.make_async_copy(y_hbm.at[sidx.at[pl.ds(g * G, G)]],
                                  rows[b], sems[b]).wait()
            pltpu.make_async_copy(dst2d_hbm.at[pl.ds(wid * GPW + g, 1)],
                                  didx[b], dsems[b]).wait()
            pltpu.sync_copy(rows[b], sacc.at[didx[b].at[0]], add=True)
        return _
    lax.fori_loop(0, GPW // 2, _pair, None)
    plsc.subcore_barrier()

    def _wb(t, _):
        base = sid * ROWS_PER_TILE + t * G
        pltpu.sync_copy(sacc.at[pl.ds(base, G)], rows0)
        pltpu.sync_copy(rows0, out_hbm.at[c, pl.ds(base, G)])
        return _
    lax.fori_loop(0, ROWS_PER_TILE // G, _wb, None)


EPW = E // NW
HGRP = (EPW + G - 1) // G


@functools.partial(
    pl.kernel, mesh=_mesh(),
    out_type=jax.ShapeDtypeStruct((E * DE,), f32),
    scratch_types=[
        pltpu.VMEM((EPW,), i32),
        pltpu.VMEM((EPW,), i32),
        pltpu.VMEM((G, D), f32),
        pltpu.VMEM((G, D), f32),
        pltpu.VMEM((G, D), f32),
        pltpu.VMEM((G, D), f32),
        pltpu.VMEM((G * DE,), f32),
        pltpu.SemaphoreType.DMA,
        pltpu.SemaphoreType.DMA,
        pltpu.SemaphoreType.DMA,
        pltpu.SemaphoreType.DMA,
    ],
)
def _sc_head(tab_hbm, src_hbm, dst_hbm, out_hbm, sidx, didx,
             abuf0, abuf1, bbuf0, bbuf1, ob, sa0, sa1, sb0, sb1):
    wid = _worker_id()

    pltpu.sync_copy(src_hbm.at[pl.ds(wid * EPW, EPW)], sidx)
    pltpu.sync_copy(dst_hbm.at[pl.ds(wid * EPW, EPW)], didx)

    abufs, bbufs = (abuf0, abuf1), (bbuf0, bbuf1)
    sas, sbs = (sa0, sa1), (sb0, sb1)

    def _gather(g, b):
        off = lax.min(g * G, EPW - G)
        pltpu.async_copy(tab_hbm.at[sidx.at[pl.ds(off, G)]], abufs[b], sas[b])
        pltpu.async_copy(tab_hbm.at[didx.at[pl.ds(off, G)]], bbufs[b], sbs[b])

    _gather(0, 0)

    def _odd(g, b):
        off = lax.min(g * G, EPW - G)

        @pl.when(g + 1 < HGRP)
        def _():
            _gather(g + 1, 1 - b)
        pltpu.make_async_copy(tab_hbm.at[sidx.at[pl.ds(off, G)]],
                              abufs[b], sas[b]).wait()
        pltpu.make_async_copy(tab_hbm.at[didx.at[pl.ds(off, G)]],
                              bbufs[b], sbs[b]).wait()

        ab, bb = abufs[b], bbufs[b]

        def _add(j, _2):
            ob[pl.ds(j * DE, DE)] = ab[j, pl.ds(0, DE)] + bb[j, pl.ds(DE, DE)]
            return _2
        lax.fori_loop(0, G, _add, None)
        base = wid * EPW + off
        pltpu.sync_copy(ob, out_hbm.at[pl.ds(base * DE, G * DE)])

    def _pair(p, _):
        _odd(p * 2, 0)
        _odd(p * 2 + 1, 1)
        return _
    lax.fori_loop(0, HGRP // 2, _pair, None)
    _odd(HGRP - 1, (HGRP - 1) % 2)


BLK = 512
GRID = NPAD // BLK


def _tca_body(xp_ref, w1_ref, xw_ref):
    xw_ref[...] = jnp.dot(xp_ref[...], w1_ref[...], preferred_element_type=f32)


def _tca(xp, w1):
    return pl.pallas_call(
        _tca_body,
        grid=(GRID,),
        in_specs=[
            pl.BlockSpec((BLK, D), lambda i: (i, 0)),
            pl.BlockSpec((D, D), lambda i: (0, 0)),
        ],
        out_specs=pl.BlockSpec((BLK, D), lambda i: (i, 0)),
        out_shape=jax.ShapeDtypeStruct((NPAD, D), f32),
    )(xp, w1)


def _tcb_body(xw_ref, degp_ref, dinv_ref, y1_ref):
    deg = 1.0 + degp_ref[0, :] + degp_ref[1, :]
    dinv = lax.rsqrt(deg)
    dinv_ref[...] = dinv
    y1_ref[...] = xw_ref[...] * dinv[:, None]


def _tcb(xw, degp):
    return pl.pallas_call(
        _tcb_body,
        grid=(GRID,),
        in_specs=[
            pl.BlockSpec((BLK, D), lambda i: (i, 0)),
            pl.BlockSpec((NC, BLK), lambda i: (0, i)),
        ],
        out_specs=[
            pl.BlockSpec((BLK,), lambda i: (i,)),
            pl.BlockSpec((BLK, D), lambda i: (i, 0)),
        ],
        out_shape=[
            jax.ShapeDtypeStruct((NPAD,), f32),
            jax.ShapeDtypeStruct((NPAD, D), f32),
        ],
    )(xw, degp)


def _tc2_body(y_ref, acc_ref, dinv_ref, b_ref, w_ref, y2_ref):
    i = pl.program_id(0)
    dinv = dinv_ref[...]
    h = (acc_ref[0] + acc_ref[1] + y_ref[...]) * dinv[:, None] + b_ref[...][None, :]
    h = jnp.maximum(h, 0.0)
    rows = lax.broadcasted_iota(i32, (BLK, 1), 0) + i * BLK
    h = jnp.where(rows < N, h, 0.0)
    y2_ref[...] = jnp.dot(h, w_ref[...], preferred_element_type=f32) * dinv[:, None]


def _tc2(y1, acc, dinv, b, w):
    return pl.pallas_call(
        _tc2_body,
        grid=(GRID,),
        in_specs=[
            pl.BlockSpec((BLK, D), lambda i: (i, 0)),
            pl.BlockSpec((NC, BLK, D), lambda i: (0, i, 0)),
            pl.BlockSpec((BLK,), lambda i: (i,)),
            pl.BlockSpec((D,), lambda i: (0,)),
            pl.BlockSpec((D, D), lambda i: (0, 0)),
        ],
        out_specs=pl.BlockSpec((BLK, D), lambda i: (i, 0)),
        out_shape=jax.ShapeDtypeStruct((NPAD, D), f32),
    )(y1, acc, dinv, b, w)


def _tc3_body(y_ref, acc_ref, dinv_ref, b_ref, wc_ref, bc_ref, tab_ref):
    i = pl.program_id(0)
    dinv = dinv_ref[...]
    h = (acc_ref[0] + acc_ref[1] + y_ref[...]) * dinv[:, None] + b_ref[...][None, :]
    h = jnp.maximum(h, 0.0)
    rows = lax.broadcasted_iota(i32, (BLK, 1), 0) + i * BLK
    h = jnp.where(rows < N, h, 0.0)
    hw = jnp.dot(h, wc_ref[...], preferred_element_type=f32) + bc_ref[...][None, :]
    tab_ref[...] = jnp.concatenate(
        [hw, jnp.zeros((BLK, D - 2 * DE), f32)], axis=1)


def _tc3(y2, acc, dinv, b, wc, bc):
    return pl.pallas_call(
        _tc3_body,
        grid=(GRID,),
        in_specs=[
            pl.BlockSpec((BLK, D), lambda i: (i, 0)),
            pl.BlockSpec((NC, BLK, D), lambda i: (0, i, 0)),
            pl.BlockSpec((BLK,), lambda i: (i,)),
            pl.BlockSpec((D,), lambda i: (0,)),
            pl.BlockSpec((D, 2 * DE), lambda i: (0, 0)),
            pl.BlockSpec((2 * DE,), lambda i: (0,)),
        ],
        out_specs=pl.BlockSpec((BLK, D), lambda i: (i, 0)),
        out_shape=jax.ShapeDtypeStruct((NPAD, D), f32),
    )(y2, acc, dinv, b, wc, bc)


def kernel(x, edge_index, edge_attr, W1, b1, W2, b2, Wl, bl):
    del edge_attr
    ei = edge_index.astype(i32)
    src, dst = ei[0], ei[1]
    pad = jnp.full((EPAD - E,), N, dtype=i32)
    srcp = jnp.concatenate([src, pad])
    dst2d = jnp.concatenate([dst, pad]).reshape(EPAD // G, G)
    xp = jnp.pad(x, ((0, NPAD - N), (0, 0)))

    degp = _sc_degree(dst2d)
    xw1 = _tca(xp, W1)
    dinv, y1 = _tcb(xw1, degp)
    acc1 = _sc_scatter(y1, srcp, dst2d)
    y2 = _tc2(y1, acc1, dinv, b1, W2)
    acc2 = _sc_scatter(y2, srcp, dst2d)
    wc = jnp.concatenate([Wl[:D], Wl[D:]], axis=1)
    bc = jnp.concatenate([bl, jnp.zeros((DE,), f32)])
    tab = _tc3(y2, acc2, dinv, b2, wc, bc)
    out_flat = _sc_head(tab, src, dst)
    return out_flat.reshape(E, DE)
